# baseline (speedup 1.0000x reference)
import functools
import math

import jax
import jax.numpy as jnp
from jax import lax
from jax.experimental import pallas as pl
from jax.experimental.pallas import tpu as pltpu
from jax.experimental.pallas import tpu_sc as plsc

N = 10000
E = 320000
D = 128
NC = 2
NS = 16
NW = NC * NS
NPAD = 10112
ROWS_PER_TILE = NPAD // NS
DUMP = N
E_PER_TILE = E // NW
K = 128
NBUF = 2
OUTERS = 39
TAIL = 16
GROUPS = K // 16

_E1 = math.exp(1.0)
_A_P = (_E1 + 1.0) / (_E1 - 1.0)
_C_P = -1.0 / (_E1 - 1.0)


def _mesh():
    return plsc.VectorSubcoreMesh(
        core_axis_name="c", subcore_axis_name="s",
        num_cores=NC, num_subcores=NS)


def _deg_body(row_hbm, col_hbm, zeros_hbm, ones_hbm, deg_out,
              row_v, col_v, rr_v, ones_v, trr_v, sem_i, sem_s, acc):
    cid = lax.axis_index("c")
    sid = lax.axis_index("s")
    wid = cid * NS + sid
    r0 = sid * ROWS_PER_TILE
    pltpu.sync_copy(zeros_hbm.at[pl.ds(r0, ROWS_PER_TILE)],
                    acc.at[pl.ds(r0, ROWS_PER_TILE)])
    pltpu.sync_copy(ones_hbm, ones_v)
    plsc.subcore_barrier()
    base0 = wid * E_PER_TILE

    def outer(o, carry):
        @pl.when(o > 0)
        def _():
            for j in range(NBUF):
                pltpu.make_async_copy(ones_v, acc.at[rr_v.at[j]], sem_s).wait()

        for j in range(NBUF):
            b = base0 + (o * NBUF + j) * K
            pltpu.async_copy(row_hbm.at[pl.ds(b, K)], row_v.at[j], sem_i)
            pltpu.async_copy(col_hbm.at[pl.ds(b, K)], col_v.at[j], sem_i)
        for j in range(NBUF):
            pltpu.make_async_copy(row_hbm.at[pl.ds(0, K)], row_v.at[j],
                                  sem_i).wait()
            pltpu.make_async_copy(col_hbm.at[pl.ds(0, K)], col_v.at[j],
                                  sem_i).wait()
        for j in range(NBUF):
            for g in range(GROUPS):
                r = row_v[j, pl.ds(g * 16, 16)]
                c = col_v[j, pl.ds(g * 16, 16)]
                rr_v[j, pl.ds(g * 16, 16)] = jnp.where(r == c, DUMP, r)
            pltpu.async_copy(ones_v, acc.at[rr_v.at[j]], sem_s, add=True)
        return carry

    lax.fori_loop(0, OUTERS, outer, 0)
    for j in range(NBUF):
        pltpu.make_async_copy(ones_v, acc.at[rr_v.at[j]], sem_s).wait()

    tb = base0 + OUTERS * NBUF * K
    pltpu.sync_copy(row_hbm.at[pl.ds(tb, TAIL)], trr_v)
    pltpu.sync_copy(col_hbm.at[pl.ds(tb, TAIL)], col_v.at[0, pl.ds(0, TAIL)])
    r = trr_v[pl.ds(0, 16)]
    c = col_v[0, pl.ds(0, 16)]
    trr_v[pl.ds(0, 16)] = jnp.where(r == c, DUMP, r)
    pltpu.sync_copy(ones_v.at[pl.ds(0, TAIL)], acc.at[trr_v], add=True)

    plsc.subcore_barrier()
    pltpu.sync_copy(acc.at[pl.ds(r0, ROWS_PER_TILE)],
                    deg_out.at[cid, pl.ds(r0, ROWS_PER_TILE)])


def _spmm_body(src_hbm, row_hbm, col_hbm, zeros_hbm, out_hbm,
               row_v, colp_v, rows_v, trow_v, tcol_v, trows_v,
               sem_i, sem_g0, sem_g1, sem_g2, sem_g3,
               sem_s0, sem_s1, sem_s2, sem_s3, acc):
    cid = lax.axis_index("c")
    sid = lax.axis_index("s")
    wid = cid * NS + sid
    r0 = sid * ROWS_PER_TILE
    pltpu.sync_copy(zeros_hbm.at[pl.ds(r0, ROWS_PER_TILE)],
                    acc.at[pl.ds(r0, ROWS_PER_TILE)])
    plsc.subcore_barrier()
    base0 = wid * E_PER_TILE
    sem_g = (sem_g0, sem_g1, sem_g2, sem_g3)[:NBUF]
    sem_s = (sem_s0, sem_s1, sem_s2, sem_s3)[:NBUF]

    for j in range(NBUF):
        b0 = base0 + j * K
        pltpu.async_copy(row_hbm.at[pl.ds(b0, K)], row_v.at[0, j], sem_i)
        pltpu.async_copy(col_hbm.at[pl.ds(b0, K)], colp_v.at[0, j], sem_i)

    def outer(o, carry):
        s = o % 2
        for j in range(NBUF):
            pltpu.make_async_copy(row_hbm.at[pl.ds(0, K)], row_v.at[s, j],
                                  sem_i).wait()
            pltpu.make_async_copy(col_hbm.at[pl.ds(0, K)], colp_v.at[s, j],
                                  sem_i).wait()
        for j in range(NBUF):
            for g in range(GROUPS):
                r = row_v[s, j, pl.ds(g * 16, 16)]
                c = colp_v[s, j, pl.ds(g * 16, 16)]
                colp_v[s, j, pl.ds(g * 16, 16)] = jnp.where(r == c, DUMP, c)
        for j in range(NBUF):
            @pl.when(o >= 1)
            def _():
                pltpu.make_async_copy(rows_v.at[j],
                                      acc.at[colp_v.at[1 - s, j]],
                                      sem_s[j]).wait()

            @pl.when(o + 1 < OUTERS)
            def _():
                b = base0 + ((o + 1) * NBUF + j) * K
                pltpu.async_copy(row_hbm.at[pl.ds(b, K)],
                                 row_v.at[1 - s, j], sem_i)
                pltpu.async_copy(col_hbm.at[pl.ds(b, K)],
                                 colp_v.at[1 - s, j], sem_i)

            pltpu.async_copy(src_hbm.at[row_v.at[s, j]], rows_v.at[j],
                             sem_g[j])
        for j in range(NBUF):
            pltpu.make_async_copy(src_hbm.at[row_v.at[s, j]], rows_v.at[j],
                                  sem_g[j]).wait()
            pltpu.async_copy(rows_v.at[j], acc.at[colp_v.at[s, j]], sem_s[j],
                             add=True)
        return carry

    lax.fori_loop(0, OUTERS, outer, 0)
    s_last = (OUTERS - 1) % 2
    for j in range(NBUF):
        pltpu.make_async_copy(rows_v.at[j], acc.at[colp_v.at[s_last, j]],
                              sem_s[j]).wait()

    tb = base0 + OUTERS * NBUF * K
    pltpu.sync_copy(row_hbm.at[pl.ds(tb, TAIL)], trow_v)
    pltpu.sync_copy(col_hbm.at[pl.ds(tb, TAIL)], tcol_v)
    r = trow_v[pl.ds(0, 16)]
    c = tcol_v[pl.ds(0, 16)]
    tcol_v[pl.ds(0, 16)] = jnp.where(r == c, DUMP, c)
    pltpu.async_copy(src_hbm.at[trow_v], trows_v, sem_g0).wait()
    pltpu.sync_copy(trows_v, acc.at[tcol_v], add=True)

    plsc.subcore_barrier()
    pltpu.sync_copy(acc.at[pl.ds(r0, ROWS_PER_TILE)],
                    out_hbm.at[cid, pl.ds(r0, ROWS_PER_TILE)])


@functools.lru_cache(maxsize=None)
def _make_deg_kernel():
    return pl.kernel(
        _deg_body,
        out_type=jax.ShapeDtypeStruct((NC, NPAD, D), jnp.float32),
        mesh=_mesh(),
        scratch_types=[
            pltpu.VMEM((NBUF, K), jnp.int32),
            pltpu.VMEM((NBUF, K), jnp.int32),
            pltpu.VMEM((NBUF, K), jnp.int32),
            pltpu.VMEM((K, D), jnp.float32),
            pltpu.VMEM((TAIL,), jnp.int32),
            pltpu.SemaphoreType.DMA,
            pltpu.SemaphoreType.DMA,
            pltpu.VMEM_SHARED((NPAD, D), jnp.float32),
        ],
    )


@functools.lru_cache(maxsize=None)
def _make_spmm_kernel():
    return pl.kernel(
        _spmm_body,
        out_type=jax.ShapeDtypeStruct((NC, NPAD, D), jnp.float32),
        mesh=_mesh(),
        scratch_types=[
            pltpu.VMEM((2, NBUF, K), jnp.int32),
            pltpu.VMEM((2, NBUF, K), jnp.int32),
            pltpu.VMEM((NBUF, K, D), jnp.float32),
            pltpu.VMEM((TAIL,), jnp.int32),
            pltpu.VMEM((TAIL,), jnp.int32),
            pltpu.VMEM((TAIL, D), jnp.float32),
            pltpu.SemaphoreType.DMA,
            pltpu.SemaphoreType.DMA,
            pltpu.SemaphoreType.DMA,
            pltpu.SemaphoreType.DMA,
            pltpu.SemaphoreType.DMA,
            pltpu.SemaphoreType.DMA,
            pltpu.SemaphoreType.DMA,
            pltpu.SemaphoreType.DMA,
            pltpu.SemaphoreType.DMA,
            pltpu.VMEM_SHARED((NPAD, D), jnp.float32),
        ],
    )


_B = 1000


def _tc1_body(degp_ref, x_ref, pm_ref, w1t_ref, u_ref, dis_ref):
    deg = degp_ref[0, :, 0:1] + degp_ref[1, :, 0:1] + 1.0
    dis = lax.rsqrt(deg)
    pm = pm_ref[...]
    a = 1.0 + pm * (_A_P - 1.0)
    c = pm * _C_P
    y = a * x_ref[...] + c
    t1 = jnp.dot(y, w1t_ref[...], preferred_element_type=jnp.float32)
    u_ref[...] = dis * t1
    dis_ref[...] = dis


def _tc2_body(p_ref, u_ref, dis_ref, b1_ref, w2t_ref, v_ref):
    agg = p_ref[0] + p_ref[1] + u_ref[...]
    dis = dis_ref[...]
    h = jnp.maximum(dis * agg + b1_ref[...], 0.0)
    v_ref[...] = jnp.dot(dis * h, w2t_ref[...],
                         preferred_element_type=jnp.float32)


def _tc3_body(p_ref, v_ref, dis_ref, b2_ref, o_ref):
    o_ref[...] = dis_ref[...] * (p_ref[0] + p_ref[1] + v_ref[...]) + b2_ref[...]


def kernel(x, edge_index, priv_mask, W1, b1, W2, b2):
    row = edge_index[0]
    col = edge_index[1]
    pm = priv_mask.astype(jnp.float32)
    w1t = W1.T
    w2t = W2.T
    b1r = b1.reshape(1, D)
    b2r = b2.reshape(1, D)
    ones_src = jnp.ones((K, D), jnp.float32)
    zeros_d = jnp.zeros((NPAD, D), jnp.float32)

    degp = _make_deg_kernel()(row, col, zeros_d, ones_src)

    u, dis = pl.pallas_call(
        _tc1_body,
        grid=(N // _B,),
        in_specs=[
            pl.BlockSpec((NC, _B, D), lambda i: (0, i, 0)),
            pl.BlockSpec((_B, D), lambda i: (i, 0)),
            pl.BlockSpec((_B, 1), lambda i: (i, 0)),
            pl.BlockSpec((D, D), lambda i: (0, 0)),
        ],
        out_specs=[
            pl.BlockSpec((_B, D), lambda i: (i, 0)),
            pl.BlockSpec((_B, 1), lambda i: (i, 0)),
        ],
        out_shape=[
            jax.ShapeDtypeStruct((N, D), jnp.float32),
            jax.ShapeDtypeStruct((N, 1), jnp.float32),
        ],
    )(degp, x, pm, w1t)

    p1 = _make_spmm_kernel()(u, row, col, zeros_d)

    v = pl.pallas_call(
        _tc2_body,
        grid=(N // _B,),
        in_specs=[
            pl.BlockSpec((NC, _B, D), lambda i: (0, i, 0)),
            pl.BlockSpec((_B, D), lambda i: (i, 0)),
            pl.BlockSpec((_B, 1), lambda i: (i, 0)),
            pl.BlockSpec((1, D), lambda i: (0, 0)),
            pl.BlockSpec((D, D), lambda i: (0, 0)),
        ],
        out_specs=pl.BlockSpec((_B, D), lambda i: (i, 0)),
        out_shape=jax.ShapeDtypeStruct((N, D), jnp.float32),
    )(p1, u, dis, b1r, w2t)

    p2 = _make_spmm_kernel()(v, row, col, zeros_d)

    out = pl.pallas_call(
        _tc3_body,
        grid=(N // _B,),
        in_specs=[
            pl.BlockSpec((NC, _B, D), lambda i: (0, i, 0)),
            pl.BlockSpec((_B, D), lambda i: (i, 0)),
            pl.BlockSpec((_B, 1), lambda i: (i, 0)),
            pl.BlockSpec((1, D), lambda i: (0, 0)),
        ],
        out_specs=pl.BlockSpec((_B, D), lambda i: (i, 0)),
        out_shape=jax.ShapeDtypeStruct((N, D), jnp.float32),
    )(p2, v, dis, b2r)

    return out


# speedup vs baseline: 1.1712x; 1.1712x over previous
import functools
import math

import jax
import jax.numpy as jnp
from jax import lax
from jax.experimental import pallas as pl
from jax.experimental.pallas import tpu as pltpu
from jax.experimental.pallas import tpu_sc as plsc

N = 10000
E = 320000
D = 128
NC = 2
NS = 16
NW = NC * NS
NPAD = 10112
ROWS_PER_TILE = NPAD // NS
DUMP = N
E_PER_TILE = E // NW
K = 80
NBUF = 4
OUTERS = 31
TAIL = K
GROUPS = K // 16

_E1 = math.exp(1.0)
_A_P = (_E1 + 1.0) / (_E1 - 1.0)
_C_P = -1.0 / (_E1 - 1.0)


def _mesh():
    return plsc.VectorSubcoreMesh(
        core_axis_name="c", subcore_axis_name="s",
        num_cores=NC, num_subcores=NS)


def _deg_body(row_hbm, col_hbm, zeros_hbm, ones_hbm, deg_out,
              row_v, col_v, rr_v, ones_v, sem_i, sem_s, acc):
    cid = lax.axis_index("c")
    sid = lax.axis_index("s")
    wid = cid * NS + sid
    r0 = sid * ROWS_PER_TILE
    pltpu.sync_copy(zeros_hbm.at[pl.ds(r0, ROWS_PER_TILE)],
                    acc.at[pl.ds(r0, ROWS_PER_TILE)])
    pltpu.sync_copy(ones_hbm, ones_v)
    plsc.subcore_barrier()
    base0 = wid * E_PER_TILE

    def outer(o, carry):
        @pl.when(o > 0)
        def _():
            for j in range(NBUF):
                pltpu.make_async_copy(ones_v, acc.at[rr_v.at[j]], sem_s).wait()

        for j in range(NBUF):
            b = base0 + (o * NBUF + j) * K
            pltpu.async_copy(row_hbm.at[pl.ds(b, K)], row_v.at[j], sem_i)
            pltpu.async_copy(col_hbm.at[pl.ds(b, K)], col_v.at[j], sem_i)
        for j in range(NBUF):
            pltpu.make_async_copy(row_hbm.at[pl.ds(0, K)], row_v.at[j],
                                  sem_i).wait()
            pltpu.make_async_copy(col_hbm.at[pl.ds(0, K)], col_v.at[j],
                                  sem_i).wait()
        for j in range(NBUF):
            for g in range(GROUPS):
                r = row_v[j, pl.ds(g * 16, 16)]
                c = col_v[j, pl.ds(g * 16, 16)]
                rr_v[j, pl.ds(g * 16, 16)] = jnp.where(r == c, DUMP, r)
            pltpu.async_copy(ones_v, acc.at[rr_v.at[j]], sem_s, add=True)
        return carry

    lax.fori_loop(0, OUTERS, outer, 0)
    for j in range(NBUF):
        pltpu.make_async_copy(ones_v, acc.at[rr_v.at[j]], sem_s).wait()

    tb = base0 + OUTERS * NBUF * K
    pltpu.sync_copy(row_hbm.at[pl.ds(tb, K)], row_v.at[0])
    pltpu.sync_copy(col_hbm.at[pl.ds(tb, K)], col_v.at[0])
    for g in range(GROUPS):
        r = row_v[0, pl.ds(g * 16, 16)]
        c = col_v[0, pl.ds(g * 16, 16)]
        rr_v[0, pl.ds(g * 16, 16)] = jnp.where(r == c, DUMP, r)
    pltpu.sync_copy(ones_v, acc.at[rr_v.at[0]], add=True)

    plsc.subcore_barrier()
    pltpu.sync_copy(acc.at[pl.ds(r0, ROWS_PER_TILE)],
                    deg_out.at[cid, pl.ds(r0, ROWS_PER_TILE)])


def _spmm_body(src_hbm, row_hbm, col_hbm, zeros_hbm, out_hbm,
               row_v, colp_v, rows_v,
               sem_i, sem_g0, sem_g1, sem_g2, sem_g3,
               sem_s0, sem_s1, sem_s2, sem_s3, acc):
    cid = lax.axis_index("c")
    sid = lax.axis_index("s")
    wid = cid * NS + sid
    r0 = sid * ROWS_PER_TILE
    pltpu.sync_copy(zeros_hbm.at[pl.ds(r0, ROWS_PER_TILE)],
                    acc.at[pl.ds(r0, ROWS_PER_TILE)])
    plsc.subcore_barrier()
    base0 = wid * E_PER_TILE
    sem_g = (sem_g0, sem_g1, sem_g2, sem_g3)[:NBUF]
    sem_s = (sem_s0, sem_s1, sem_s2, sem_s3)[:NBUF]

    for j in range(NBUF):
        b0 = base0 + j * K
        pltpu.async_copy(row_hbm.at[pl.ds(b0, K)], row_v.at[0, j], sem_i)
        pltpu.async_copy(col_hbm.at[pl.ds(b0, K)], colp_v.at[0, j], sem_i)

    def outer(o, carry):
        s = o % 2
        for j in range(NBUF):
            pltpu.make_async_copy(row_hbm.at[pl.ds(0, K)], row_v.at[s, j],
                                  sem_i).wait()
            pltpu.make_async_copy(col_hbm.at[pl.ds(0, K)], colp_v.at[s, j],
                                  sem_i).wait()
        for j in range(NBUF):
            for g in range(GROUPS):
                r = row_v[s, j, pl.ds(g * 16, 16)]
                c = colp_v[s, j, pl.ds(g * 16, 16)]
                colp_v[s, j, pl.ds(g * 16, 16)] = jnp.where(r == c, DUMP, c)
        for j in range(NBUF):
            @pl.when(o >= 1)
            def _():
                pltpu.make_async_copy(rows_v.at[j],
                                      acc.at[colp_v.at[1 - s, j]],
                                      sem_s[j]).wait()

            @pl.when(o + 1 < OUTERS)
            def _():
                b = base0 + ((o + 1) * NBUF + j) * K
                pltpu.async_copy(row_hbm.at[pl.ds(b, K)],
                                 row_v.at[1 - s, j], sem_i)
                pltpu.async_copy(col_hbm.at[pl.ds(b, K)],
                                 colp_v.at[1 - s, j], sem_i)

            pltpu.async_copy(src_hbm.at[row_v.at[s, j]], rows_v.at[j],
                             sem_g[j])
        for j in range(NBUF):
            pltpu.make_async_copy(src_hbm.at[row_v.at[s, j]], rows_v.at[j],
                                  sem_g[j]).wait()
            pltpu.async_copy(rows_v.at[j], acc.at[colp_v.at[s, j]], sem_s[j],
                             add=True)
        return carry

    lax.fori_loop(0, OUTERS, outer, 0)
    s_last = (OUTERS - 1) % 2
    for j in range(NBUF):
        pltpu.make_async_copy(rows_v.at[j], acc.at[colp_v.at[s_last, j]],
                              sem_s[j]).wait()

    tb = base0 + OUTERS * NBUF * K
    pltpu.sync_copy(row_hbm.at[pl.ds(tb, K)], row_v.at[0, 0])
    pltpu.sync_copy(col_hbm.at[pl.ds(tb, K)], colp_v.at[0, 0])
    for g in range(GROUPS):
        r = row_v[0, 0, pl.ds(g * 16, 16)]
        c = colp_v[0, 0, pl.ds(g * 16, 16)]
        colp_v[0, 0, pl.ds(g * 16, 16)] = jnp.where(r == c, DUMP, c)
    pltpu.async_copy(src_hbm.at[row_v.at[0, 0]], rows_v.at[0], sem_g0).wait()
    pltpu.sync_copy(rows_v.at[0], acc.at[colp_v.at[0, 0]], add=True)

    plsc.subcore_barrier()
    pltpu.sync_copy(acc.at[pl.ds(r0, ROWS_PER_TILE)],
                    out_hbm.at[cid, pl.ds(r0, ROWS_PER_TILE)])


@functools.lru_cache(maxsize=None)
def _make_deg_kernel():
    return pl.kernel(
        _deg_body,
        out_type=jax.ShapeDtypeStruct((NC, NPAD, D), jnp.float32),
        mesh=_mesh(),
        scratch_types=[
            pltpu.VMEM((NBUF, K), jnp.int32),
            pltpu.VMEM((NBUF, K), jnp.int32),
            pltpu.VMEM((NBUF, K), jnp.int32),
            pltpu.VMEM((K, D), jnp.float32),
            pltpu.SemaphoreType.DMA,
            pltpu.SemaphoreType.DMA,
            pltpu.VMEM_SHARED((NPAD, D), jnp.float32),
        ],
    )


@functools.lru_cache(maxsize=None)
def _make_spmm_kernel():
    return pl.kernel(
        _spmm_body,
        out_type=jax.ShapeDtypeStruct((NC, NPAD, D), jnp.float32),
        mesh=_mesh(),
        scratch_types=[
            pltpu.VMEM((2, NBUF, K), jnp.int32),
            pltpu.VMEM((2, NBUF, K), jnp.int32),
            pltpu.VMEM((NBUF, K, D), jnp.float32),
            pltpu.SemaphoreType.DMA,
            pltpu.SemaphoreType.DMA,
            pltpu.SemaphoreType.DMA,
            pltpu.SemaphoreType.DMA,
            pltpu.SemaphoreType.DMA,
            pltpu.SemaphoreType.DMA,
            pltpu.SemaphoreType.DMA,
            pltpu.SemaphoreType.DMA,
            pltpu.SemaphoreType.DMA,
            pltpu.VMEM_SHARED((NPAD, D), jnp.float32),
        ],
    )


_B = 1000


def _tc1_body(degp_ref, x_ref, pm_ref, w1t_ref, u_ref, dis_ref):
    deg = degp_ref[0, :, 0:1] + degp_ref[1, :, 0:1] + 1.0
    dis = lax.rsqrt(deg)
    pm = pm_ref[...]
    a = 1.0 + pm * (_A_P - 1.0)
    c = pm * _C_P
    y = a * x_ref[...] + c
    t1 = jnp.dot(y, w1t_ref[...], preferred_element_type=jnp.float32)
    u_ref[...] = dis * t1
    dis_ref[...] = dis


def _tc2_body(p_ref, u_ref, dis_ref, b1_ref, w2t_ref, v_ref):
    agg = p_ref[0] + p_ref[1] + u_ref[...]
    dis = dis_ref[...]
    h = jnp.maximum(dis * agg + b1_ref[...], 0.0)
    v_ref[...] = jnp.dot(dis * h, w2t_ref[...],
                         preferred_element_type=jnp.float32)


def _tc3_body(p_ref, v_ref, dis_ref, b2_ref, o_ref):
    o_ref[...] = dis_ref[...] * (p_ref[0] + p_ref[1] + v_ref[...]) + b2_ref[...]


def kernel(x, edge_index, priv_mask, W1, b1, W2, b2):
    row = edge_index[0]
    col = edge_index[1]
    pm = priv_mask.astype(jnp.float32)
    w1t = W1.T
    w2t = W2.T
    b1r = b1.reshape(1, D)
    b2r = b2.reshape(1, D)
    ones_src = jnp.ones((K, D), jnp.float32)
    zeros_d = jnp.zeros((NPAD, D), jnp.float32)

    degp = _make_deg_kernel()(row, col, zeros_d, ones_src)

    u, dis = pl.pallas_call(
        _tc1_body,
        grid=(N // _B,),
        in_specs=[
            pl.BlockSpec((NC, _B, D), lambda i: (0, i, 0)),
            pl.BlockSpec((_B, D), lambda i: (i, 0)),
            pl.BlockSpec((_B, 1), lambda i: (i, 0)),
            pl.BlockSpec((D, D), lambda i: (0, 0)),
        ],
        out_specs=[
            pl.BlockSpec((_B, D), lambda i: (i, 0)),
            pl.BlockSpec((_B, 1), lambda i: (i, 0)),
        ],
        out_shape=[
            jax.ShapeDtypeStruct((N, D), jnp.float32),
            jax.ShapeDtypeStruct((N, 1), jnp.float32),
        ],
    )(degp, x, pm, w1t)

    p1 = _make_spmm_kernel()(u, row, col, zeros_d)

    v = pl.pallas_call(
        _tc2_body,
        grid=(N // _B,),
        in_specs=[
            pl.BlockSpec((NC, _B, D), lambda i: (0, i, 0)),
            pl.BlockSpec((_B, D), lambda i: (i, 0)),
            pl.BlockSpec((_B, 1), lambda i: (i, 0)),
            pl.BlockSpec((1, D), lambda i: (0, 0)),
            pl.BlockSpec((D, D), lambda i: (0, 0)),
        ],
        out_specs=pl.BlockSpec((_B, D), lambda i: (i, 0)),
        out_shape=jax.ShapeDtypeStruct((N, D), jnp.float32),
    )(p1, u, dis, b1r, w2t)

    p2 = _make_spmm_kernel()(v, row, col, zeros_d)

    out = pl.pallas_call(
        _tc3_body,
        grid=(N // _B,),
        in_specs=[
            pl.BlockSpec((NC, _B, D), lambda i: (0, i, 0)),
            pl.BlockSpec((_B, D), lambda i: (i, 0)),
            pl.BlockSpec((_B, 1), lambda i: (i, 0)),
            pl.BlockSpec((1, D), lambda i: (0, 0)),
        ],
        out_specs=pl.BlockSpec((_B, D), lambda i: (i, 0)),
        out_shape=jax.ShapeDtypeStruct((N, D), jnp.float32),
    )(p2, v, dis, b2r)

    return out


# speedup vs baseline: 1.1921x; 1.0178x over previous
import functools
import math

import jax
import jax.numpy as jnp
from jax import lax
from jax.experimental import pallas as pl
from jax.experimental.pallas import tpu as pltpu
from jax.experimental.pallas import tpu_sc as plsc

N = 10000
E = 320000
D = 128
NC = 2
NS = 16
NW = NC * NS
NPAD = 10112
ROWS_PER_TILE = NPAD // NS
DUMP = N
E_PER_TILE = E // NW
K = 80
NBUF = 4
OUTERS = 31
TAIL = K
GROUPS = K // 16

_E1 = math.exp(1.0)
_A_P = (_E1 + 1.0) / (_E1 - 1.0)
_C_P = -1.0 / (_E1 - 1.0)


def _mesh():
    return plsc.VectorSubcoreMesh(
        core_axis_name="c", subcore_axis_name="s",
        num_cores=NC, num_subcores=NS)


def _deg_body(row_hbm, col_hbm, zeros_hbm, ones_hbm, deg_out,
              row_v, col_v, rr_v, ones_v, sem_i, sem_s, acc):
    cid = lax.axis_index("c")
    sid = lax.axis_index("s")
    wid = cid * NS + sid
    r0 = sid * ROWS_PER_TILE
    pltpu.sync_copy(zeros_hbm.at[pl.ds(r0, ROWS_PER_TILE)],
                    acc.at[pl.ds(r0, ROWS_PER_TILE)])
    pltpu.sync_copy(ones_hbm, ones_v)
    plsc.subcore_barrier()
    base0 = wid * E_PER_TILE

    def outer(o, carry):
        @pl.when(o > 0)
        def _():
            for j in range(NBUF):
                pltpu.make_async_copy(ones_v, acc.at[rr_v.at[j]], sem_s).wait()

        for j in range(NBUF):
            b = base0 + (o * NBUF + j) * K
            pltpu.async_copy(row_hbm.at[pl.ds(b, K)], row_v.at[j], sem_i)
            pltpu.async_copy(col_hbm.at[pl.ds(b, K)], col_v.at[j], sem_i)
        for j in range(NBUF):
            pltpu.make_async_copy(row_hbm.at[pl.ds(0, K)], row_v.at[j],
                                  sem_i).wait()
            pltpu.make_async_copy(col_hbm.at[pl.ds(0, K)], col_v.at[j],
                                  sem_i).wait()
        for j in range(NBUF):
            for g in range(GROUPS):
                r = row_v[j, pl.ds(g * 16, 16)]
                c = col_v[j, pl.ds(g * 16, 16)]
                rr_v[j, pl.ds(g * 16, 16)] = jnp.where(r == c, DUMP, r)
            pltpu.async_copy(ones_v, acc.at[rr_v.at[j]], sem_s, add=True)
        return carry

    lax.fori_loop(0, OUTERS, outer, 0)
    for j in range(NBUF):
        pltpu.make_async_copy(ones_v, acc.at[rr_v.at[j]], sem_s).wait()

    tb = base0 + OUTERS * NBUF * K
    pltpu.sync_copy(row_hbm.at[pl.ds(tb, K)], row_v.at[0])
    pltpu.sync_copy(col_hbm.at[pl.ds(tb, K)], col_v.at[0])
    for g in range(GROUPS):
        r = row_v[0, pl.ds(g * 16, 16)]
        c = col_v[0, pl.ds(g * 16, 16)]
        rr_v[0, pl.ds(g * 16, 16)] = jnp.where(r == c, DUMP, r)
    pltpu.sync_copy(ones_v, acc.at[rr_v.at[0]], add=True)

    plsc.subcore_barrier()
    pltpu.sync_copy(acc.at[pl.ds(r0, ROWS_PER_TILE)],
                    deg_out.at[cid, pl.ds(r0, ROWS_PER_TILE)])


def _spmm_body(src_hbm, row_hbm, col_hbm, zeros_hbm, out_hbm,
               row_v, colp_v, rows_v,
               sem_i, sem_g0, sem_g1, sem_g2, sem_g3,
               sem_s0, sem_s1, sem_s2, sem_s3, acc):
    cid = lax.axis_index("c")
    sid = lax.axis_index("s")
    wid = cid * NS + sid
    r0 = sid * ROWS_PER_TILE
    base0 = wid * E_PER_TILE
    sem_g = (sem_g0, sem_g1, sem_g2, sem_g3)[:NBUF]
    sem_s = (sem_s0, sem_s1, sem_s2, sem_s3)[:NBUF]

    for j in range(NBUF):
        b0 = base0 + j * K
        pltpu.async_copy(row_hbm.at[pl.ds(b0, K)], row_v.at[0, j], sem_i)
        pltpu.async_copy(col_hbm.at[pl.ds(b0, K)], colp_v.at[0, j], sem_i)

    pltpu.sync_copy(zeros_hbm.at[pl.ds(r0, ROWS_PER_TILE)],
                    acc.at[pl.ds(r0, ROWS_PER_TILE)])
    plsc.subcore_barrier()

    def outer(o, carry):
        s = o % 2
        for j in range(NBUF):
            pltpu.make_async_copy(row_hbm.at[pl.ds(0, K)], row_v.at[s, j],
                                  sem_i).wait()
            pltpu.make_async_copy(col_hbm.at[pl.ds(0, K)], colp_v.at[s, j],
                                  sem_i).wait()
        for j in range(NBUF):
            for g in range(GROUPS):
                r = row_v[s, j, pl.ds(g * 16, 16)]
                c = colp_v[s, j, pl.ds(g * 16, 16)]
                colp_v[s, j, pl.ds(g * 16, 16)] = jnp.where(r == c, DUMP, c)
        for j in range(NBUF):
            @pl.when(o >= 1)
            def _():
                pltpu.make_async_copy(rows_v.at[j],
                                      acc.at[colp_v.at[1 - s, j]],
                                      sem_s[j]).wait()

            @pl.when(o + 1 < OUTERS)
            def _():
                b = base0 + ((o + 1) * NBUF + j) * K
                pltpu.async_copy(row_hbm.at[pl.ds(b, K)],
                                 row_v.at[1 - s, j], sem_i)
                pltpu.async_copy(col_hbm.at[pl.ds(b, K)],
                                 colp_v.at[1 - s, j], sem_i)

            pltpu.async_copy(src_hbm.at[row_v.at[s, j]], rows_v.at[j],
                             sem_g[j])
        for j in range(NBUF):
            pltpu.make_async_copy(src_hbm.at[row_v.at[s, j]], rows_v.at[j],
                                  sem_g[j]).wait()
            pltpu.async_copy(rows_v.at[j], acc.at[colp_v.at[s, j]], sem_s[j],
                             add=True)
        return carry

    lax.fori_loop(0, OUTERS, outer, 0)
    s_last = (OUTERS - 1) % 2
    for j in range(NBUF):
        pltpu.make_async_copy(rows_v.at[j], acc.at[colp_v.at[s_last, j]],
                              sem_s[j]).wait()

    tb = base0 + OUTERS * NBUF * K
    pltpu.sync_copy(row_hbm.at[pl.ds(tb, K)], row_v.at[0, 0])
    pltpu.sync_copy(col_hbm.at[pl.ds(tb, K)], colp_v.at[0, 0])
    for g in range(GROUPS):
        r = row_v[0, 0, pl.ds(g * 16, 16)]
        c = colp_v[0, 0, pl.ds(g * 16, 16)]
        colp_v[0, 0, pl.ds(g * 16, 16)] = jnp.where(r == c, DUMP, c)
    pltpu.async_copy(src_hbm.at[row_v.at[0, 0]], rows_v.at[0], sem_g0).wait()
    pltpu.sync_copy(rows_v.at[0], acc.at[colp_v.at[0, 0]], add=True)

    plsc.subcore_barrier()
    pltpu.sync_copy(acc.at[pl.ds(r0, ROWS_PER_TILE)],
                    out_hbm.at[cid, pl.ds(r0, ROWS_PER_TILE)])


@functools.lru_cache(maxsize=None)
def _make_deg_kernel():
    return pl.kernel(
        _deg_body,
        out_type=jax.ShapeDtypeStruct((NC, NPAD, D), jnp.float32),
        mesh=_mesh(),
        scratch_types=[
            pltpu.VMEM((NBUF, K), jnp.int32),
            pltpu.VMEM((NBUF, K), jnp.int32),
            pltpu.VMEM((NBUF, K), jnp.int32),
            pltpu.VMEM((K, D), jnp.float32),
            pltpu.SemaphoreType.DMA,
            pltpu.SemaphoreType.DMA,
            pltpu.VMEM_SHARED((NPAD, D), jnp.float32),
        ],
    )


@functools.lru_cache(maxsize=None)
def _make_spmm_kernel():
    return pl.kernel(
        _spmm_body,
        out_type=jax.ShapeDtypeStruct((NC, NPAD, D), jnp.float32),
        mesh=_mesh(),
        scratch_types=[
            pltpu.VMEM((2, NBUF, K), jnp.int32),
            pltpu.VMEM((2, NBUF, K), jnp.int32),
            pltpu.VMEM((NBUF, K, D), jnp.float32),
            pltpu.SemaphoreType.DMA,
            pltpu.SemaphoreType.DMA,
            pltpu.SemaphoreType.DMA,
            pltpu.SemaphoreType.DMA,
            pltpu.SemaphoreType.DMA,
            pltpu.SemaphoreType.DMA,
            pltpu.SemaphoreType.DMA,
            pltpu.SemaphoreType.DMA,
            pltpu.SemaphoreType.DMA,
            pltpu.VMEM_SHARED((NPAD, D), jnp.float32),
        ],
    )


_B = 2000


def _tc1_body(degp_ref, x_ref, pm_ref, w1t_ref, u_ref, dis_ref):
    deg = degp_ref[0, :, 0:1] + degp_ref[1, :, 0:1] + 1.0
    dis = lax.rsqrt(deg)
    pm = pm_ref[...]
    a = 1.0 + pm * (_A_P - 1.0)
    c = pm * _C_P
    y = a * x_ref[...] + c
    t1 = jnp.dot(y, w1t_ref[...], preferred_element_type=jnp.float32)
    u_ref[...] = dis * t1
    dis_ref[...] = dis


def _tc2_body(p_ref, u_ref, dis_ref, b1_ref, w2t_ref, v_ref):
    agg = p_ref[0] + p_ref[1] + u_ref[...]
    dis = dis_ref[...]
    h = jnp.maximum(dis * agg + b1_ref[...], 0.0)
    v_ref[...] = jnp.dot(dis * h, w2t_ref[...],
                         preferred_element_type=jnp.float32)


def _tc3_body(p_ref, v_ref, dis_ref, b2_ref, o_ref):
    o_ref[...] = dis_ref[...] * (p_ref[0] + p_ref[1] + v_ref[...]) + b2_ref[...]


def kernel(x, edge_index, priv_mask, W1, b1, W2, b2):
    row = edge_index[0]
    col = edge_index[1]
    pm = priv_mask.astype(jnp.float32)
    w1t = W1.T
    w2t = W2.T
    b1r = b1.reshape(1, D)
    b2r = b2.reshape(1, D)
    ones_src = jnp.ones((K, D), jnp.float32)
    zeros_d = jnp.zeros((NPAD, D), jnp.float32)

    degp = _make_deg_kernel()(row, col, zeros_d, ones_src)

    u, dis = pl.pallas_call(
        _tc1_body,
        grid=(N // _B,),
        in_specs=[
            pl.BlockSpec((NC, _B, D), lambda i: (0, i, 0)),
            pl.BlockSpec((_B, D), lambda i: (i, 0)),
            pl.BlockSpec((_B, 1), lambda i: (i, 0)),
            pl.BlockSpec((D, D), lambda i: (0, 0)),
        ],
        out_specs=[
            pl.BlockSpec((_B, D), lambda i: (i, 0)),
            pl.BlockSpec((_B, 1), lambda i: (i, 0)),
        ],
        out_shape=[
            jax.ShapeDtypeStruct((N, D), jnp.float32),
            jax.ShapeDtypeStruct((N, 1), jnp.float32),
        ],
    )(degp, x, pm, w1t)

    p1 = _make_spmm_kernel()(u, row, col, zeros_d)

    v = pl.pallas_call(
        _tc2_body,
        grid=(N // _B,),
        in_specs=[
            pl.BlockSpec((NC, _B, D), lambda i: (0, i, 0)),
            pl.BlockSpec((_B, D), lambda i: (i, 0)),
            pl.BlockSpec((_B, 1), lambda i: (i, 0)),
            pl.BlockSpec((1, D), lambda i: (0, 0)),
            pl.BlockSpec((D, D), lambda i: (0, 0)),
        ],
        out_specs=pl.BlockSpec((_B, D), lambda i: (i, 0)),
        out_shape=jax.ShapeDtypeStruct((N, D), jnp.float32),
    )(p1, u, dis, b1r, w2t)

    p2 = _make_spmm_kernel()(v, row, col, zeros_d)

    out = pl.pallas_call(
        _tc3_body,
        grid=(N // _B,),
        in_specs=[
            pl.BlockSpec((NC, _B, D), lambda i: (0, i, 0)),
            pl.BlockSpec((_B, D), lambda i: (i, 0)),
            pl.BlockSpec((_B, 1), lambda i: (i, 0)),
            pl.BlockSpec((1, D), lambda i: (0, 0)),
        ],
        out_specs=pl.BlockSpec((_B, D), lambda i: (i, 0)),
        out_shape=jax.ShapeDtypeStruct((N, D), jnp.float32),
    )(p2, v, dis, b2r)

    return out


# speedup vs baseline: 1.1932x; 1.0010x over previous
import functools
import math

import jax
import jax.numpy as jnp
from jax import lax
from jax.experimental import pallas as pl
from jax.experimental.pallas import tpu as pltpu
from jax.experimental.pallas import tpu_sc as plsc

N = 10000
E = 320000
D = 128
NC = 2
NS = 16
NW = NC * NS
NPAD = 10112
ROWS_PER_TILE = NPAD // NS
DUMP = N
E_PER_TILE = E // NW
K = 80
NBUF = 4
OUTERS = 31
TAIL = K
GROUPS = K // 16

_E1 = math.exp(1.0)
_A_P = (_E1 + 1.0) / (_E1 - 1.0)
_C_P = -1.0 / (_E1 - 1.0)


def _mesh():
    return plsc.VectorSubcoreMesh(
        core_axis_name="c", subcore_axis_name="s",
        num_cores=NC, num_subcores=NS)


def _deg_body(row_hbm, col_hbm, zeros_hbm, ones_hbm, deg_out,
              row_v, col_v, rr_v, ones_v, sem_i, sem_s, acc):
    cid = lax.axis_index("c")
    sid = lax.axis_index("s")
    wid = cid * NS + sid
    r0 = sid * ROWS_PER_TILE
    pltpu.sync_copy(zeros_hbm.at[pl.ds(r0, ROWS_PER_TILE)],
                    acc.at[pl.ds(r0, ROWS_PER_TILE)])
    pltpu.sync_copy(ones_hbm, ones_v)
    plsc.subcore_barrier()
    base0 = wid * E_PER_TILE

    def outer(o, carry):
        @pl.when(o > 0)
        def _():
            for j in range(NBUF):
                pltpu.make_async_copy(ones_v, acc.at[rr_v.at[j]], sem_s).wait()

        for j in range(NBUF):
            b = base0 + (o * NBUF + j) * K
            pltpu.async_copy(row_hbm.at[pl.ds(b, K)], row_v.at[j], sem_i)
            pltpu.async_copy(col_hbm.at[pl.ds(b, K)], col_v.at[j], sem_i)
        for j in range(NBUF):
            pltpu.make_async_copy(row_hbm.at[pl.ds(0, K)], row_v.at[j],
                                  sem_i).wait()
            pltpu.make_async_copy(col_hbm.at[pl.ds(0, K)], col_v.at[j],
                                  sem_i).wait()
        for j in range(NBUF):
            for g in range(GROUPS):
                r = row_v[j, pl.ds(g * 16, 16)]
                c = col_v[j, pl.ds(g * 16, 16)]
                rr_v[j, pl.ds(g * 16, 16)] = jnp.where(r == c, DUMP, r)
            pltpu.async_copy(ones_v, acc.at[rr_v.at[j]], sem_s, add=True)
        return carry

    lax.fori_loop(0, OUTERS, outer, 0)
    for j in range(NBUF):
        pltpu.make_async_copy(ones_v, acc.at[rr_v.at[j]], sem_s).wait()

    tb = base0 + OUTERS * NBUF * K
    pltpu.sync_copy(row_hbm.at[pl.ds(tb, K)], row_v.at[0])
    pltpu.sync_copy(col_hbm.at[pl.ds(tb, K)], col_v.at[0])
    for g in range(GROUPS):
        r = row_v[0, pl.ds(g * 16, 16)]
        c = col_v[0, pl.ds(g * 16, 16)]
        rr_v[0, pl.ds(g * 16, 16)] = jnp.where(r == c, DUMP, r)
    pltpu.sync_copy(ones_v, acc.at[rr_v.at[0]], add=True)

    plsc.subcore_barrier()
    pltpu.sync_copy(acc.at[pl.ds(r0, ROWS_PER_TILE)],
                    deg_out.at[cid, pl.ds(r0, ROWS_PER_TILE)])


def _spmm_body(src_hbm, row_hbm, col_hbm, zeros_hbm, out_hbm,
               row_v, colp_v, rows_v,
               sem_i, sem_g0, sem_g1, sem_g2, sem_g3,
               sem_s0, sem_s1, sem_s2, sem_s3, acc):
    cid = lax.axis_index("c")
    sid = lax.axis_index("s")
    wid = cid * NS + sid
    r0 = sid * ROWS_PER_TILE
    base0 = wid * E_PER_TILE
    sem_g = (sem_g0, sem_g1, sem_g2, sem_g3)[:NBUF]
    sem_s = (sem_s0, sem_s1, sem_s2, sem_s3)[:NBUF]

    for j in range(NBUF):
        b0 = base0 + j * K
        pltpu.async_copy(row_hbm.at[pl.ds(b0, K)], row_v.at[0, j], sem_i)
        pltpu.async_copy(col_hbm.at[pl.ds(b0, K)], colp_v.at[0, j], sem_i)

    pltpu.sync_copy(zeros_hbm.at[pl.ds(r0, ROWS_PER_TILE)],
                    acc.at[pl.ds(r0, ROWS_PER_TILE)])
    plsc.subcore_barrier()

    def outer(o, carry):
        s = o % 2
        for j in range(NBUF):
            pltpu.make_async_copy(row_hbm.at[pl.ds(0, K)], row_v.at[s, j],
                                  sem_i).wait()
            pltpu.make_async_copy(col_hbm.at[pl.ds(0, K)], colp_v.at[s, j],
                                  sem_i).wait()
        for j in range(NBUF):
            for g in range(GROUPS):
                r = row_v[s, j, pl.ds(g * 16, 16)]
                c = colp_v[s, j, pl.ds(g * 16, 16)]
                colp_v[s, j, pl.ds(g * 16, 16)] = jnp.where(r == c, DUMP, c)
        for j in range(NBUF):
            @pl.when(o >= 1)
            def _():
                pltpu.make_async_copy(rows_v.at[j],
                                      acc.at[colp_v.at[1 - s, j]],
                                      sem_s[j]).wait()

            @pl.when(o + 1 < OUTERS)
            def _():
                b = base0 + ((o + 1) * NBUF + j) * K
                pltpu.async_copy(row_hbm.at[pl.ds(b, K)],
                                 row_v.at[1 - s, j], sem_i)
                pltpu.async_copy(col_hbm.at[pl.ds(b, K)],
                                 colp_v.at[1 - s, j], sem_i)

            pltpu.async_copy(src_hbm.at[row_v.at[s, j]], rows_v.at[j],
                             sem_g[j])
        for j in range(NBUF):
            pltpu.make_async_copy(src_hbm.at[row_v.at[s, j]], rows_v.at[j],
                                  sem_g[j]).wait()
            pltpu.async_copy(rows_v.at[j], acc.at[colp_v.at[s, j]], sem_s[j],
                             add=True)
        return carry

    lax.fori_loop(0, OUTERS, outer, 0)
    s_last = (OUTERS - 1) % 2
    for j in range(NBUF):
        pltpu.make_async_copy(rows_v.at[j], acc.at[colp_v.at[s_last, j]],
                              sem_s[j]).wait()

    tb = base0 + OUTERS * NBUF * K
    pltpu.sync_copy(row_hbm.at[pl.ds(tb, K)], row_v.at[0, 0])
    pltpu.sync_copy(col_hbm.at[pl.ds(tb, K)], colp_v.at[0, 0])
    for g in range(GROUPS):
        r = row_v[0, 0, pl.ds(g * 16, 16)]
        c = colp_v[0, 0, pl.ds(g * 16, 16)]
        colp_v[0, 0, pl.ds(g * 16, 16)] = jnp.where(r == c, DUMP, c)
    pltpu.async_copy(src_hbm.at[row_v.at[0, 0]], rows_v.at[0], sem_g0).wait()
    pltpu.sync_copy(rows_v.at[0], acc.at[colp_v.at[0, 0]], add=True)

    plsc.subcore_barrier()
    pltpu.sync_copy(acc.at[pl.ds(r0, ROWS_PER_TILE)],
                    out_hbm.at[cid, pl.ds(r0, ROWS_PER_TILE)])


@functools.lru_cache(maxsize=None)
def _make_deg_kernel():
    return pl.kernel(
        _deg_body,
        out_type=jax.ShapeDtypeStruct((NC, NPAD, D), jnp.float32),
        mesh=_mesh(),
        scratch_types=[
            pltpu.VMEM((NBUF, K), jnp.int32),
            pltpu.VMEM((NBUF, K), jnp.int32),
            pltpu.VMEM((NBUF, K), jnp.int32),
            pltpu.VMEM((K, D), jnp.float32),
            pltpu.SemaphoreType.DMA,
            pltpu.SemaphoreType.DMA,
            pltpu.VMEM_SHARED((NPAD, D), jnp.float32),
        ],
    )


@functools.lru_cache(maxsize=None)
def _make_spmm_kernel():
    return pl.kernel(
        _spmm_body,
        out_type=jax.ShapeDtypeStruct((NC, NPAD, D), jnp.float32),
        mesh=_mesh(),
        scratch_types=[
            pltpu.VMEM((2, NBUF, K), jnp.int32),
            pltpu.VMEM((2, NBUF, K), jnp.int32),
            pltpu.VMEM((NBUF, K, D), jnp.float32),
            pltpu.SemaphoreType.DMA,
            pltpu.SemaphoreType.DMA,
            pltpu.SemaphoreType.DMA,
            pltpu.SemaphoreType.DMA,
            pltpu.SemaphoreType.DMA,
            pltpu.SemaphoreType.DMA,
            pltpu.SemaphoreType.DMA,
            pltpu.SemaphoreType.DMA,
            pltpu.SemaphoreType.DMA,
            pltpu.VMEM_SHARED((NPAD, D), jnp.float32),
        ],
    )


_B = 2000


def _tc1a_body(x_ref, pm_ref, w1t_ref, t1_ref):
    pm = pm_ref[...]
    a = 1.0 + pm * (_A_P - 1.0)
    c = pm * _C_P
    y = a * x_ref[...] + c
    t1_ref[...] = jnp.dot(y, w1t_ref[...], preferred_element_type=jnp.float32)


def _tc1b_body(degp_ref, t1_ref, u_ref, dis_ref):
    deg = degp_ref[0, :, 0:1] + degp_ref[1, :, 0:1] + 1.0
    dis = lax.rsqrt(deg)
    u_ref[...] = dis * t1_ref[...]
    dis_ref[...] = dis


def _tc2_body(p_ref, u_ref, dis_ref, b1_ref, w2t_ref, v_ref):
    agg = p_ref[0] + p_ref[1] + u_ref[...]
    dis = dis_ref[...]
    h = jnp.maximum(dis * agg + b1_ref[...], 0.0)
    v_ref[...] = jnp.dot(dis * h, w2t_ref[...],
                         preferred_element_type=jnp.float32)


def _tc3_body(p_ref, v_ref, dis_ref, b2_ref, o_ref):
    o_ref[...] = dis_ref[...] * (p_ref[0] + p_ref[1] + v_ref[...]) + b2_ref[...]


def kernel(x, edge_index, priv_mask, W1, b1, W2, b2):
    row = edge_index[0]
    col = edge_index[1]
    pm = priv_mask.astype(jnp.float32)
    w1t = W1.T
    w2t = W2.T
    b1r = b1.reshape(1, D)
    b2r = b2.reshape(1, D)
    ones_src = jnp.ones((K, D), jnp.float32)
    zeros_d = jnp.zeros((NPAD, D), jnp.float32)

    degp = _make_deg_kernel()(row, col, zeros_d, ones_src)

    t1 = pl.pallas_call(
        _tc1a_body,
        grid=(N // _B,),
        in_specs=[
            pl.BlockSpec((_B, D), lambda i: (i, 0)),
            pl.BlockSpec((_B, 1), lambda i: (i, 0)),
            pl.BlockSpec((D, D), lambda i: (0, 0)),
        ],
        out_specs=pl.BlockSpec((_B, D), lambda i: (i, 0)),
        out_shape=jax.ShapeDtypeStruct((N, D), jnp.float32),
    )(x, pm, w1t)

    u, dis = pl.pallas_call(
        _tc1b_body,
        grid=(N // _B,),
        in_specs=[
            pl.BlockSpec((NC, _B, D), lambda i: (0, i, 0)),
            pl.BlockSpec((_B, D), lambda i: (i, 0)),
        ],
        out_specs=[
            pl.BlockSpec((_B, D), lambda i: (i, 0)),
            pl.BlockSpec((_B, 1), lambda i: (i, 0)),
        ],
        out_shape=[
            jax.ShapeDtypeStruct((N, D), jnp.float32),
            jax.ShapeDtypeStruct((N, 1), jnp.float32),
        ],
    )(degp, t1)

    p1 = _make_spmm_kernel()(u, row, col, zeros_d)

    v = pl.pallas_call(
        _tc2_body,
        grid=(N // _B,),
        in_specs=[
            pl.BlockSpec((NC, _B, D), lambda i: (0, i, 0)),
            pl.BlockSpec((_B, D), lambda i: (i, 0)),
            pl.BlockSpec((_B, 1), lambda i: (i, 0)),
            pl.BlockSpec((1, D), lambda i: (0, 0)),
            pl.BlockSpec((D, D), lambda i: (0, 0)),
        ],
        out_specs=pl.BlockSpec((_B, D), lambda i: (i, 0)),
        out_shape=jax.ShapeDtypeStruct((N, D), jnp.float32),
    )(p1, u, dis, b1r, w2t)

    p2 = _make_spmm_kernel()(v, row, col, zeros_d)

    out = pl.pallas_call(
        _tc3_body,
        grid=(N // _B,),
        in_specs=[
            pl.BlockSpec((NC, _B, D), lambda i: (0, i, 0)),
            pl.BlockSpec((_B, D), lambda i: (i, 0)),
            pl.BlockSpec((_B, 1), lambda i: (i, 0)),
            pl.BlockSpec((1, D), lambda i: (0, 0)),
        ],
        out_specs=pl.BlockSpec((_B, D), lambda i: (i, 0)),
        out_shape=jax.ShapeDtypeStruct((N, D), jnp.float32),
    )(p2, v, dis, b2r)

    return out
